# baseline (speedup 1.0000x reference)
import jax
import jax.numpy as jnp
from jax.experimental import pallas as pl

_DIM = 8192
_B = 32
_K = 8
_TILE = 256
_NT = _DIM // _TILE

_NEG_INF = float("-inf")


def _fused_kernel(x_ref, w_ref, vals_ref, idx_ref):
    t = pl.program_id(0)

    @pl.when(t == 0)
    def _init():
        vals_ref[...] = jnp.full((_B, _K), _NEG_INF, jnp.float32)
        idx_ref[...] = jnp.zeros((_B, _K), jnp.int32)

    y = jax.lax.dot_general(
        x_ref[...], w_ref[...],
        (((1,), (1,)), ((), ())),
        preferred_element_type=jnp.float32,
    )

    base = t * _TILE
    col = jax.lax.broadcasted_iota(jnp.int32, (_B, _TILE), 1) + base

    cand_v = jnp.concatenate([vals_ref[...], y], axis=1)
    cand_i = jnp.concatenate([idx_ref[...], col], axis=1)
    pos = jax.lax.broadcasted_iota(jnp.int32, cand_v.shape, 1)

    new_v = []
    new_i = []
    for _ in range(_K):
        m = jnp.max(cand_v, axis=-1, keepdims=True)
        a = jnp.argmax(cand_v, axis=-1).astype(jnp.int32)
        a = a[:, None]
        hit = pos == a
        sel_i = jnp.sum(jnp.where(hit, cand_i, 0), axis=-1, keepdims=True)
        new_v.append(m)
        new_i.append(sel_i)
        cand_v = jnp.where(hit, _NEG_INF, cand_v)

    vals_ref[...] = jnp.concatenate(new_v, axis=1)
    idx_ref[...] = jnp.concatenate(new_i, axis=1)


def kernel(x, W):
    vals, idx = pl.pallas_call(
        _fused_kernel,
        grid=(_NT,),
        in_specs=[
            pl.BlockSpec((_B, _DIM), lambda i: (0, 0)),
            pl.BlockSpec((_TILE, _DIM), lambda i: (i, 0)),
        ],
        out_specs=[
            pl.BlockSpec((_B, _K), lambda i: (0, 0)),
            pl.BlockSpec((_B, _K), lambda i: (0, 0)),
        ],
        out_shape=[
            jax.ShapeDtypeStruct((_B, _K), jnp.float32),
            jax.ShapeDtypeStruct((_B, _K), jnp.int32),
        ],
    )(x, W)
    return (vals, idx)


# speedup vs baseline: 1.2246x; 1.2246x over previous
import jax
import jax.numpy as jnp
from jax.experimental import pallas as pl

_DIM = 8192
_B = 32
_K = 8
_TILE = 512
_NT = _DIM // _TILE

_NEG_INF = float("-inf")


def _fused_kernel(x_ref, w_ref, vals_ref, idx_ref):
    t = pl.program_id(0)

    @pl.when(t == 0)
    def _init():
        vals_ref[...] = jnp.full((_B, _K), _NEG_INF, jnp.float32)
        idx_ref[...] = jnp.zeros((_B, _K), jnp.int32)

    y = jax.lax.dot_general(
        x_ref[...], w_ref[...],
        (((1,), (1,)), ((), ())),
        preferred_element_type=jnp.float32,
    )

    base = t * _TILE
    col = jax.lax.broadcasted_iota(jnp.int32, (_B, _TILE), 1) + base

    cand_v = jnp.concatenate([vals_ref[...], y], axis=1)
    cand_i = jnp.concatenate([idx_ref[...], col], axis=1)
    pos = jax.lax.broadcasted_iota(jnp.int32, cand_v.shape, 1)

    new_v = []
    new_i = []
    for _ in range(_K):
        m = jnp.max(cand_v, axis=-1, keepdims=True)
        a = jnp.argmax(cand_v, axis=-1).astype(jnp.int32)
        a = a[:, None]
        hit = pos == a
        sel_i = jnp.sum(jnp.where(hit, cand_i, 0), axis=-1, keepdims=True)
        new_v.append(m)
        new_i.append(sel_i)
        cand_v = jnp.where(hit, _NEG_INF, cand_v)

    vals_ref[...] = jnp.concatenate(new_v, axis=1)
    idx_ref[...] = jnp.concatenate(new_i, axis=1)


def kernel(x, W):
    vals, idx = pl.pallas_call(
        _fused_kernel,
        grid=(_NT,),
        in_specs=[
            pl.BlockSpec((_B, _DIM), lambda i: (0, 0)),
            pl.BlockSpec((_TILE, _DIM), lambda i: (i, 0)),
        ],
        out_specs=[
            pl.BlockSpec((_B, _K), lambda i: (0, 0)),
            pl.BlockSpec((_B, _K), lambda i: (0, 0)),
        ],
        out_shape=[
            jax.ShapeDtypeStruct((_B, _K), jnp.float32),
            jax.ShapeDtypeStruct((_B, _K), jnp.int32),
        ],
    )(x, W)
    return (vals, idx)
